# baseline (speedup 1.0000x reference)
import functools

import jax
import jax.numpy as jnp
import numpy as np
from jax.experimental import pallas as pl
from jax.experimental.pallas import tpu as pltpu

MAX_LEN = 24
BATCH = 8
VOCAB = 16000
EMB = 256
HID = 512
S_ENC = MAX_LEN - 1
ROWS = MAX_LEN * BATCH
CAT = HID + HID + EMB

VCH = 640
NCH = VOCAB // VCH

_PREC = jax.lax.Precision.DEFAULT


def _dot(a, b):
    return jnp.dot(a, b, precision=_PREC, preferred_element_type=jnp.float32)


def _dot_t(a, w):
    return jax.lax.dot_general(a, w, (((1,), (1,)), ((), ())),
                               precision=_PREC,
                               preferred_element_type=jnp.float32)


def _lstm(x, h, c, wx, wh, b):
    g = _dot_t(x, wx) + _dot_t(h, wh) + b
    i = g[:, 0:HID]
    f = g[:, HID:2 * HID]
    gg = g[:, 2 * HID:3 * HID]
    o = g[:, 3 * HID:4 * HID]
    c2 = jax.nn.sigmoid(f) * c + jax.nn.sigmoid(i) * jnp.tanh(gg)
    h2 = jax.nn.sigmoid(o) * jnp.tanh(c2)
    return h2, c2


def _recur_kernel(src_ref, enc_emb, dec_emb,
                  ew0x, ew0h, eb0, ew1x, ew1h, eb1,
                  dw0x, dw0h, db0, dw1x, dw1h, db1,
                  cat_ref, enc_out_ref, rows_ref, sem):
    def issue_enc(i, _):
        t = i // BATCH
        b = i % BATCH
        pltpu.make_async_copy(
            enc_emb.at[pl.ds(src_ref[S_ENC - t, b], 1), :],
            rows_ref.at[pl.ds(i, 1), :], sem).start()
        return 0

    def issue_dec(i, _):
        t = i // BATCH
        b = i % BATCH
        pltpu.make_async_copy(
            dec_emb.at[pl.ds(src_ref[t, b], 1), :],
            rows_ref.at[pl.ds(S_ENC * BATCH + i, 1), :], sem).start()
        return 0

    def wait_one(i, _):
        pltpu.make_async_copy(
            enc_emb.at[pl.ds(0, 1), :], rows_ref.at[pl.ds(0, 1), :],
            sem).wait()
        return 0

    jax.lax.fori_loop(0, S_ENC * BATCH, issue_enc, 0)
    jax.lax.fori_loop(0, MAX_LEN * BATCH, issue_dec, 0)
    jax.lax.fori_loop(0, (S_ENC + MAX_LEN) * BATCH, wait_one, 0)

    zero = jnp.zeros((BATCH, HID), jnp.float32)

    def enc_body(t, carry):
        h0, c0, h1, c1 = carry
        x = rows_ref[pl.ds(t * BATCH, BATCH), :]
        h0, c0 = _lstm(x, h0, c0, ew0x[:, :], ew0h[:, :], eb0[:, :])
        h1, c1 = _lstm(h0, h1, c1, ew1x[:, :], ew1h[:, :], eb1[:, :])
        enc_out_ref[pl.ds(t, 1)] = h1[None]
        return (h0, c0, h1, c1)

    carry = jax.lax.fori_loop(0, S_ENC, enc_body, (zero, zero, zero, zero))

    inv_sqrt_h = np.float32(1.0 / np.sqrt(HID))

    def dec_body(t, carry):
        h0, c0, h1, c1 = carry
        emb = rows_ref[pl.ds((S_ENC + t) * BATCH, BATCH), :]
        enc_out = enc_out_ref[:, :, :]
        scores = jnp.sum(enc_out * h1[None], axis=2) * inv_sqrt_h
        m = jnp.max(scores, axis=0, keepdims=True)
        e = jnp.exp(scores - m)
        attn = e / jnp.sum(e, axis=0, keepdims=True)
        ctx = jnp.sum(attn[:, :, None] * enc_out, axis=0)
        x = jnp.concatenate([emb, ctx], axis=1)
        h0, c0 = _lstm(x, h0, c0, dw0x[:, :], dw0h[:, :], db0[:, :])
        h1, c1 = _lstm(h0, h1, c1, dw1x[:, :], dw1h[:, :], db1[:, :])
        base = t * BATCH
        cat_ref[pl.ds(base, BATCH), 0:HID] = h1
        cat_ref[pl.ds(base, BATCH), HID:2 * HID] = ctx
        cat_ref[pl.ds(base, BATCH), 2 * HID:CAT] = emb
        return (h0, c0, h1, c1)

    jax.lax.fori_loop(0, MAX_LEN, dec_body, carry)


def _proj_kernel(catt_ref, w_ref, b_ref, outt_ref, m_ref, s_ref):
    j = pl.program_id(0)
    logits = _dot_t(catt_ref[:, :], w_ref[:, :]) + b_ref[:, :]
    outt_ref[:, pl.ds(j * VCH, VCH)] = logits

    @pl.when(j == 0)
    def _():
        m_ref[:, :] = jnp.full((ROWS, 1), -jnp.inf, jnp.float32)
        s_ref[:, :] = jnp.zeros((ROWS, 1), jnp.float32)

    m = m_ref[:, :]
    mj = jnp.max(logits, axis=1, keepdims=True)
    m2 = jnp.maximum(m, mj)
    s_ref[:, :] = (s_ref[:, :] * jnp.exp(m - m2)
                   + jnp.sum(jnp.exp(logits - m2), axis=1, keepdims=True))
    m_ref[:, :] = m2

    @pl.when(j == NCH - 1)
    def _():
        lse = m_ref[:, :] + jnp.log(s_ref[:, :])

        def pass2(k, _):
            sl = pl.ds(k * VCH, VCH)
            outt_ref[:, sl] = outt_ref[:, sl] - lse
            return 0

        jax.lax.fori_loop(0, NCH, pass2, 0)


@jax.jit
def _run(src, params):
    p = params
    f32 = jnp.float32
    cat = pl.pallas_call(
        _recur_kernel,
        out_shape=jax.ShapeDtypeStruct((ROWS, CAT), f32),
        in_specs=[pl.BlockSpec(memory_space=pltpu.SMEM),
                  pl.BlockSpec(memory_space=pltpu.MemorySpace.HBM),
                  pl.BlockSpec(memory_space=pltpu.MemorySpace.HBM)] + [
            pl.BlockSpec(memory_space=pltpu.VMEM)] * 12,
        out_specs=pl.BlockSpec(memory_space=pltpu.VMEM),
        scratch_shapes=[pltpu.VMEM((S_ENC, BATCH, HID), f32),
                        pltpu.VMEM(((S_ENC + MAX_LEN) * BATCH, EMB), f32),
                        pltpu.SemaphoreType.DMA],
    )(
        src.astype(jnp.int32),
        p['enc_emb'].astype(f32), p['dec_emb'].astype(f32),
        p['enc_Wih_0'].astype(f32), p['enc_Whh_0'].astype(f32),
        p['enc_b_0'].reshape(1, -1).astype(f32),
        p['enc_Wih_1'].astype(f32), p['enc_Whh_1'].astype(f32),
        p['enc_b_1'].reshape(1, -1).astype(f32),
        p['dec_Wih_0'].astype(f32), p['dec_Whh_0'].astype(f32),
        p['dec_b_0'].reshape(1, -1).astype(f32),
        p['dec_Wih_1'].astype(f32), p['dec_Whh_1'].astype(f32),
        p['dec_b_1'].reshape(1, -1).astype(f32),
    )

    outt = pl.pallas_call(
        _proj_kernel,
        grid=(NCH,),
        out_shape=jax.ShapeDtypeStruct((ROWS, VOCAB), f32),
        in_specs=[
            pl.BlockSpec((ROWS, CAT), lambda j: (0, 0)),
            pl.BlockSpec((VCH, CAT), lambda j: (j, 0)),
            pl.BlockSpec((1, VCH), lambda j: (0, j)),
        ],
        out_specs=pl.BlockSpec((ROWS, VOCAB), lambda j: (0, 0)),
        scratch_shapes=[pltpu.VMEM((ROWS, 1), f32),
                        pltpu.VMEM((ROWS, 1), f32)],
        compiler_params=pltpu.CompilerParams(
            dimension_semantics=("arbitrary",)),
    )(cat, p['fc_W'].astype(f32), p['fc_b'].reshape(1, -1).astype(f32))

    return outt.reshape(MAX_LEN, BATCH, VOCAB)


def kernel(src, trg, beam_width, teacher_force, params):
    del trg, beam_width, teacher_force
    return _run(src, params)


# speedup vs baseline: 1.1652x; 1.1652x over previous
import functools

import jax
import jax.numpy as jnp
import numpy as np
from jax.experimental import pallas as pl
from jax.experimental.pallas import tpu as pltpu

MAX_LEN = 24
BATCH = 8
VOCAB = 16000
EMB = 256
HID = 512
S_ENC = MAX_LEN - 1
ROWS = MAX_LEN * BATCH
CAT = HID + HID + EMB

VCH = 640
NCH = VOCAB // VCH

_PREC = jax.lax.Precision.DEFAULT


def _dot(a, b):
    return jnp.dot(a, b, precision=_PREC, preferred_element_type=jnp.float32)


def _dot_t(a, w):
    return jax.lax.dot_general(a, w, (((1,), (1,)), ((), ())),
                               precision=_PREC,
                               preferred_element_type=jnp.float32)


def _dot_bf(a, w):
    return jax.lax.dot_general(a.astype(jnp.bfloat16), w,
                               (((1,), (0,)), ((), ())),
                               preferred_element_type=jnp.float32)


def _lstm(x, h, c, wx, wh, b):
    g = _dot_bf(x, wx) + _dot_bf(h, wh) + b
    i = g[:, 0:HID]
    f = g[:, HID:2 * HID]
    gg = g[:, 2 * HID:3 * HID]
    o = g[:, 3 * HID:4 * HID]
    c2 = jax.nn.sigmoid(f) * c + jax.nn.sigmoid(i) * jnp.tanh(gg)
    h2 = jax.nn.sigmoid(o) * jnp.tanh(c2)
    return h2, c2


def _recur_kernel(src_ref, enc_emb, dec_emb,
                  ew0x, ew0h, eb0, ew1x, ew1h, eb1,
                  dw0x, dw0h, db0, dw1x, dw1h, db1,
                  cat_ref, enc_out_ref, rows_ref, sem):
    def issue_enc(i, _):
        t = i // BATCH
        b = i % BATCH
        pltpu.make_async_copy(
            enc_emb.at[pl.ds(src_ref[S_ENC - t, b], 1), :],
            rows_ref.at[pl.ds(i, 1), :], sem).start()
        return 0

    def issue_dec(i, _):
        t = i // BATCH
        b = i % BATCH
        pltpu.make_async_copy(
            dec_emb.at[pl.ds(src_ref[t, b], 1), :],
            rows_ref.at[pl.ds(S_ENC * BATCH + i, 1), :], sem).start()
        return 0

    def wait_one(i, _):
        pltpu.make_async_copy(
            enc_emb.at[pl.ds(0, 1), :], rows_ref.at[pl.ds(0, 1), :],
            sem).wait()
        return 0

    jax.lax.fori_loop(0, S_ENC * BATCH, issue_enc, 0)
    jax.lax.fori_loop(0, MAX_LEN * BATCH, issue_dec, 0)
    jax.lax.fori_loop(0, (S_ENC + MAX_LEN) * BATCH, wait_one, 0)

    zero = jnp.zeros((BATCH, HID), jnp.float32)

    def enc_body(t, carry):
        h0, c0, h1, c1 = carry
        x = rows_ref[pl.ds(t * BATCH, BATCH), :]
        h0, c0 = _lstm(x, h0, c0, ew0x[:, :], ew0h[:, :], eb0[:, :])
        h1, c1 = _lstm(h0, h1, c1, ew1x[:, :], ew1h[:, :], eb1[:, :])
        enc_out_ref[pl.ds(t, 1)] = h1[None]
        return (h0, c0, h1, c1)

    carry = jax.lax.fori_loop(0, S_ENC, enc_body, (zero, zero, zero, zero))

    inv_sqrt_h = np.float32(1.0 / np.sqrt(HID))

    def dec_body(t, carry):
        h0, c0, h1, c1 = carry
        emb = rows_ref[pl.ds((S_ENC + t) * BATCH, BATCH), :]
        enc_out = enc_out_ref[:, :, :]
        scores = jnp.sum(enc_out * h1[None], axis=2) * inv_sqrt_h
        m = jnp.max(scores, axis=0, keepdims=True)
        e = jnp.exp(scores - m)
        attn = e / jnp.sum(e, axis=0, keepdims=True)
        ctx = jnp.sum(attn[:, :, None] * enc_out, axis=0)
        x = jnp.concatenate([emb, ctx], axis=1)
        h0, c0 = _lstm(x, h0, c0, dw0x[:, :], dw0h[:, :], db0[:, :])
        h1, c1 = _lstm(h0, h1, c1, dw1x[:, :], dw1h[:, :], db1[:, :])
        base = t * BATCH
        cat_ref[pl.ds(base, BATCH), 0:HID] = h1
        cat_ref[pl.ds(base, BATCH), HID:2 * HID] = ctx
        cat_ref[pl.ds(base, BATCH), 2 * HID:CAT] = emb
        return (h0, c0, h1, c1)

    jax.lax.fori_loop(0, MAX_LEN, dec_body, carry)


def _proj_kernel(catt_ref, w_ref, b_ref, outt_ref, m_ref, s_ref):
    j = pl.program_id(0)
    logits = _dot_t(catt_ref[:, :], w_ref[:, :]) + b_ref[:, :]
    outt_ref[:, pl.ds(j * VCH, VCH)] = logits

    @pl.when(j == 0)
    def _():
        m_ref[:, :] = jnp.full((ROWS, 1), -jnp.inf, jnp.float32)
        s_ref[:, :] = jnp.zeros((ROWS, 1), jnp.float32)

    m = m_ref[:, :]
    mj = jnp.max(logits, axis=1, keepdims=True)
    m2 = jnp.maximum(m, mj)
    s_ref[:, :] = (s_ref[:, :] * jnp.exp(m - m2)
                   + jnp.sum(jnp.exp(logits - m2), axis=1, keepdims=True))
    m_ref[:, :] = m2

    @pl.when(j == NCH - 1)
    def _():
        lse = m_ref[:, :] + jnp.log(s_ref[:, :])

        def pass2(k, _):
            sl = pl.ds(k * VCH, VCH)
            outt_ref[:, sl] = outt_ref[:, sl] - lse
            return 0

        jax.lax.fori_loop(0, NCH, pass2, 0)


@jax.jit
def _run(src, params):
    p = params
    f32 = jnp.float32
    cat = pl.pallas_call(
        _recur_kernel,
        out_shape=jax.ShapeDtypeStruct((ROWS, CAT), f32),
        in_specs=[pl.BlockSpec(memory_space=pltpu.SMEM),
                  pl.BlockSpec(memory_space=pltpu.MemorySpace.HBM),
                  pl.BlockSpec(memory_space=pltpu.MemorySpace.HBM)] + [
            pl.BlockSpec(memory_space=pltpu.VMEM)] * 12,
        out_specs=pl.BlockSpec(memory_space=pltpu.VMEM),
        scratch_shapes=[pltpu.VMEM((S_ENC, BATCH, HID), f32),
                        pltpu.VMEM(((S_ENC + MAX_LEN) * BATCH, EMB), f32),
                        pltpu.SemaphoreType.DMA],
    )(
        src.astype(jnp.int32),
        p['enc_emb'].astype(f32), p['dec_emb'].astype(f32),
        p['enc_Wih_0'].T.astype(jnp.bfloat16),
        p['enc_Whh_0'].T.astype(jnp.bfloat16),
        p['enc_b_0'].reshape(1, -1).astype(f32),
        p['enc_Wih_1'].T.astype(jnp.bfloat16),
        p['enc_Whh_1'].T.astype(jnp.bfloat16),
        p['enc_b_1'].reshape(1, -1).astype(f32),
        p['dec_Wih_0'].T.astype(jnp.bfloat16),
        p['dec_Whh_0'].T.astype(jnp.bfloat16),
        p['dec_b_0'].reshape(1, -1).astype(f32),
        p['dec_Wih_1'].T.astype(jnp.bfloat16),
        p['dec_Whh_1'].T.astype(jnp.bfloat16),
        p['dec_b_1'].reshape(1, -1).astype(f32),
    )

    outt = pl.pallas_call(
        _proj_kernel,
        grid=(NCH,),
        out_shape=jax.ShapeDtypeStruct((ROWS, VOCAB), f32),
        in_specs=[
            pl.BlockSpec((ROWS, CAT), lambda j: (0, 0)),
            pl.BlockSpec((VCH, CAT), lambda j: (j, 0)),
            pl.BlockSpec((1, VCH), lambda j: (0, j)),
        ],
        out_specs=pl.BlockSpec((ROWS, VOCAB), lambda j: (0, 0)),
        scratch_shapes=[pltpu.VMEM((ROWS, 1), f32),
                        pltpu.VMEM((ROWS, 1), f32)],
        compiler_params=pltpu.CompilerParams(
            dimension_semantics=("arbitrary",)),
    )(cat, p['fc_W'].astype(f32), p['fc_b'].reshape(1, -1).astype(f32))

    return outt.reshape(MAX_LEN, BATCH, VOCAB)


def kernel(src, trg, beam_width, teacher_force, params):
    del trg, beam_width, teacher_force
    return _run(src, params)


# speedup vs baseline: 1.2390x; 1.0633x over previous
import functools

import jax
import jax.numpy as jnp
import numpy as np
from jax.experimental import pallas as pl
from jax.experimental.pallas import tpu as pltpu

MAX_LEN = 24
BATCH = 8
VOCAB = 16000
EMB = 256
HID = 512
S_ENC = MAX_LEN - 1
ROWS = MAX_LEN * BATCH
CAT = HID + HID + EMB

VCH = 3200
NCH = VOCAB // VCH

_PREC = jax.lax.Precision.DEFAULT


def _dot(a, b):
    return jnp.dot(a, b, precision=_PREC, preferred_element_type=jnp.float32)


def _dot_t(a, w):
    return jax.lax.dot_general(a, w, (((1,), (1,)), ((), ())),
                               precision=_PREC,
                               preferred_element_type=jnp.float32)


def _dot_bf(a, w):
    return jax.lax.dot_general(a.astype(jnp.bfloat16), w,
                               (((1,), (0,)), ((), ())),
                               preferred_element_type=jnp.float32)


def _lstm(x, h, c, wx, wh, b):
    g = _dot_bf(x, wx) + _dot_bf(h, wh) + b
    i = g[:, 0:HID]
    f = g[:, HID:2 * HID]
    gg = g[:, 2 * HID:3 * HID]
    o = g[:, 3 * HID:4 * HID]
    c2 = jax.nn.sigmoid(f) * c + jax.nn.sigmoid(i) * jnp.tanh(gg)
    h2 = jax.nn.sigmoid(o) * jnp.tanh(c2)
    return h2, c2


def _recur_kernel(src_ref, enc_emb, dec_emb,
                  ew0x, ew0h, eb0, ew1x, ew1h, eb1,
                  dw0x, dw0h, db0, dw1x, dw1h, db1,
                  cat_ref, enc_out_ref, rows_ref, sem):
    def issue_enc(i, _):
        t = i // BATCH
        b = i % BATCH
        pltpu.make_async_copy(
            enc_emb.at[pl.ds(src_ref[S_ENC - t, b], 1), :],
            rows_ref.at[pl.ds(i, 1), :], sem).start()
        return 0

    def issue_dec(i, _):
        t = i // BATCH
        b = i % BATCH
        pltpu.make_async_copy(
            dec_emb.at[pl.ds(src_ref[t, b], 1), :],
            rows_ref.at[pl.ds(S_ENC * BATCH + i, 1), :], sem).start()
        return 0

    def wait_one(i, _):
        pltpu.make_async_copy(
            enc_emb.at[pl.ds(0, 1), :], rows_ref.at[pl.ds(0, 1), :],
            sem).wait()
        return 0

    jax.lax.fori_loop(0, S_ENC * BATCH, issue_enc, 0)
    jax.lax.fori_loop(0, MAX_LEN * BATCH, issue_dec, 0)
    jax.lax.fori_loop(0, (S_ENC + MAX_LEN) * BATCH, wait_one, 0)

    zero = jnp.zeros((BATCH, HID), jnp.float32)

    def enc_body(t, carry):
        h0, c0, h1, c1 = carry
        x = rows_ref[pl.ds(t * BATCH, BATCH), :]
        h0, c0 = _lstm(x, h0, c0, ew0x[:, :], ew0h[:, :], eb0[:, :])
        h1, c1 = _lstm(h0, h1, c1, ew1x[:, :], ew1h[:, :], eb1[:, :])
        enc_out_ref[pl.ds(t, 1)] = h1[None]
        return (h0, c0, h1, c1)

    carry = jax.lax.fori_loop(0, S_ENC, enc_body, (zero, zero, zero, zero))

    inv_sqrt_h = np.float32(1.0 / np.sqrt(HID))

    def dec_body(t, carry):
        h0, c0, h1, c1 = carry
        emb = rows_ref[pl.ds((S_ENC + t) * BATCH, BATCH), :]
        enc_out = enc_out_ref[:, :, :]
        scores = jnp.sum(enc_out * h1[None], axis=2) * inv_sqrt_h
        m = jnp.max(scores, axis=0, keepdims=True)
        e = jnp.exp(scores - m)
        attn = e / jnp.sum(e, axis=0, keepdims=True)
        ctx = jnp.sum(attn[:, :, None] * enc_out, axis=0)
        x = jnp.concatenate([emb, ctx], axis=1)
        h0, c0 = _lstm(x, h0, c0, dw0x[:, :], dw0h[:, :], db0[:, :])
        h1, c1 = _lstm(h0, h1, c1, dw1x[:, :], dw1h[:, :], db1[:, :])
        base = t * BATCH
        cat_ref[pl.ds(base, BATCH), 0:HID] = h1
        cat_ref[pl.ds(base, BATCH), HID:2 * HID] = ctx
        cat_ref[pl.ds(base, BATCH), 2 * HID:CAT] = emb
        return (h0, c0, h1, c1)

    jax.lax.fori_loop(0, MAX_LEN, dec_body, carry)


def _proj_kernel(catt_ref, w_ref, b_ref, outt_ref, m_ref, s_ref):
    j = pl.program_id(0)
    logits = _dot_t(catt_ref[:, :], w_ref[:, :]) + b_ref[:, :]
    outt_ref[:, pl.ds(j * VCH, VCH)] = logits

    @pl.when(j == 0)
    def _():
        m_ref[:, :] = jnp.full((ROWS, 1), -jnp.inf, jnp.float32)
        s_ref[:, :] = jnp.zeros((ROWS, 1), jnp.float32)

    m = m_ref[:, :]
    mj = jnp.max(logits, axis=1, keepdims=True)
    m2 = jnp.maximum(m, mj)
    s_ref[:, :] = (s_ref[:, :] * jnp.exp(m - m2)
                   + jnp.sum(jnp.exp(logits - m2), axis=1, keepdims=True))
    m_ref[:, :] = m2

    @pl.when(j == NCH - 1)
    def _():
        lse = m_ref[:, :] + jnp.log(s_ref[:, :])

        def pass2(k, _):
            sl = pl.ds(k * VCH, VCH)
            outt_ref[:, sl] = outt_ref[:, sl] - lse
            return 0

        jax.lax.fori_loop(0, NCH, pass2, 0)


@jax.jit
def _run(src, params):
    p = params
    f32 = jnp.float32
    cat = pl.pallas_call(
        _recur_kernel,
        out_shape=jax.ShapeDtypeStruct((ROWS, CAT), f32),
        in_specs=[pl.BlockSpec(memory_space=pltpu.SMEM),
                  pl.BlockSpec(memory_space=pltpu.MemorySpace.HBM),
                  pl.BlockSpec(memory_space=pltpu.MemorySpace.HBM)] + [
            pl.BlockSpec(memory_space=pltpu.VMEM)] * 12,
        out_specs=pl.BlockSpec(memory_space=pltpu.VMEM),
        scratch_shapes=[pltpu.VMEM((S_ENC, BATCH, HID), f32),
                        pltpu.VMEM(((S_ENC + MAX_LEN) * BATCH, EMB), f32),
                        pltpu.SemaphoreType.DMA],
    )(
        src.astype(jnp.int32),
        p['enc_emb'].astype(f32), p['dec_emb'].astype(f32),
        p['enc_Wih_0'].T.astype(jnp.bfloat16),
        p['enc_Whh_0'].T.astype(jnp.bfloat16),
        p['enc_b_0'].reshape(1, -1).astype(f32),
        p['enc_Wih_1'].T.astype(jnp.bfloat16),
        p['enc_Whh_1'].T.astype(jnp.bfloat16),
        p['enc_b_1'].reshape(1, -1).astype(f32),
        p['dec_Wih_0'].T.astype(jnp.bfloat16),
        p['dec_Whh_0'].T.astype(jnp.bfloat16),
        p['dec_b_0'].reshape(1, -1).astype(f32),
        p['dec_Wih_1'].T.astype(jnp.bfloat16),
        p['dec_Whh_1'].T.astype(jnp.bfloat16),
        p['dec_b_1'].reshape(1, -1).astype(f32),
    )

    outt = pl.pallas_call(
        _proj_kernel,
        grid=(NCH,),
        out_shape=jax.ShapeDtypeStruct((ROWS, VOCAB), f32),
        in_specs=[
            pl.BlockSpec((ROWS, CAT), lambda j: (0, 0)),
            pl.BlockSpec((VCH, CAT), lambda j: (j, 0)),
            pl.BlockSpec((1, VCH), lambda j: (0, j)),
        ],
        out_specs=pl.BlockSpec((ROWS, VOCAB), lambda j: (0, 0)),
        scratch_shapes=[pltpu.VMEM((ROWS, 1), f32),
                        pltpu.VMEM((ROWS, 1), f32)],
        compiler_params=pltpu.CompilerParams(
            dimension_semantics=("arbitrary",)),
    )(cat, p['fc_W'].astype(f32), p['fc_b'].reshape(1, -1).astype(f32))

    return outt.reshape(MAX_LEN, BATCH, VOCAB)


def kernel(src, trg, beam_width, teacher_force, params):
    del trg, beam_width, teacher_force
    return _run(src, params)
